# Initial kernel scaffold; baseline (speedup 1.0000x reference)
import jax
import jax.numpy as jnp
from jax.experimental import pallas as pl


def kernel(E_C, E_D, E_M, dst_C, dst_D, dst_M, W1, W2, W3):
    raise NotImplementedError("write your pallas kernel here")


# speedup vs baseline: 4.7896x; 4.7896x over previous
import functools

import jax
import jax.numpy as jnp
from jax import lax
from jax.experimental import pallas as pl
from jax.experimental.pallas import tpu as pltpu
from jax.experimental.pallas import tpu_sc as plsc

N_NODES = 10000
N_EDGES = 320000
D = 128

NC = 2
NS = 16
NW = NC * NS
GROUP = 128
NG = N_EDGES // GROUP
TRIPS = (NG + NW - 1) // NW
NPAD = 10240
NPT = NPAD // NS
HR = NPAD // D


def _sc_body(eC, dC, eD, dD, eM, dM,
             sC, hC, sD, hD, sM, hM,
             acc, ebuf, ibuf, hist):
    c = lax.axis_index("c")
    s = lax.axis_index("s")
    wid = s * NC + c
    row0 = s * NPT

    for (E, Dst, S_out, H_out) in ((eC, dC, sC, hC),
                                   (eD, dD, sD, hD),
                                   (eM, dM, sM, hM)):
        def _zero_row(i, carry):
            for j in range(D // 16):
                ebuf[i, pl.ds(j * 16, 16)] = jnp.zeros((16,), jnp.float32)
            return carry
        lax.fori_loop(0, GROUP, _zero_row, 0)
        for z in range(NPT // GROUP):
            pltpu.sync_copy(ebuf, acc.at[pl.ds(row0 + z * GROUP, GROUP)])

        def _zero_hist(i, carry):
            hist[pl.ds(i * 16, 16)] = jnp.zeros((16,), jnp.float32)
            return carry
        lax.fori_loop(0, NPAD // 16, _zero_hist, 0)
        plsc.subcore_barrier()

        def _grp(t, carry):
            g = wid + NW * t

            @pl.when(g < NG)
            def _():
                pltpu.sync_copy(E.at[pl.ds(g * GROUP, GROUP)], ebuf)
                pltpu.sync_copy(Dst.at[pl.ds(g * GROUP, GROUP)], ibuf)
                pltpu.sync_copy(ebuf, acc.at[ibuf], add=True)
                for j in range(GROUP // 16):
                    idx = ibuf[pl.ds(j * 16, 16)]
                    plsc.addupdate_scatter(
                        hist, [idx], jnp.ones((16,), jnp.float32))
            return carry
        lax.fori_loop(0, TRIPS, _grp, 0)
        plsc.subcore_barrier()

        for z in range(NPT // GROUP):
            r0 = row0 + z * GROUP
            pltpu.sync_copy(acc.at[pl.ds(r0, GROUP)], ebuf)
            pltpu.sync_copy(ebuf, S_out.at[c, pl.ds(r0, GROUP)])
        pltpu.sync_copy(hist, H_out.at[wid])


@jax.jit
def _sc_segment_sums(eC, dC, eD, dD, eM, dM):
    out_type = [
        jax.ShapeDtypeStruct((NC, NPAD, D), jnp.float32),
        jax.ShapeDtypeStruct((NW, NPAD), jnp.float32),
    ] * 3
    scratch = [
        pltpu.VMEM_SHARED((NPAD, D), jnp.float32),
        pltpu.VMEM((GROUP, D), jnp.float32),
        pltpu.VMEM((GROUP,), jnp.int32),
        pltpu.VMEM((NPAD,), jnp.float32),
    ]
    mesh = plsc.VectorSubcoreMesh(core_axis_name="c", subcore_axis_name="s")
    f = pl.kernel(_sc_body, out_type=out_type, mesh=mesh,
                  scratch_types=scratch,
                  compiler_params=pltpu.CompilerParams(
                      needs_layout_passes=False))
    return f(eC, dC, eD, dD, eM, dM)


def _tc_body(sC, kC, sD, kD, sM, kM, w1, w2, w3, out):
    w1v = w1[...]
    w2v = w2[...]
    w3v = w3[...]

    def seg_mean(sref, kref):
        ssum = sref[0] + sref[1]
        return ssum / jnp.maximum(kref[...], 1.0)

    yC = seg_mean(sC, kC)
    yD = seg_mean(sD, kD)
    yM = seg_mean(sM, kM)

    def att(y):
        return jnp.dot(jnp.tanh(jnp.dot(y, w1v, preferred_element_type=jnp.float32)),
                       w2v, preferred_element_type=jnp.float32)

    eC = att(yC)
    eD = att(yD)
    eM = att(yM)
    m = jnp.maximum(jnp.maximum(eC, eD), eM)
    xC = jnp.exp(eC - m)
    xD = jnp.exp(eD - m)
    xM = jnp.exp(eM - m)
    den = xC + xD + xM
    oy = (xC * yC + xD * yD + xM * yM) / den
    out[...] = jnp.dot(oy, w3v, preferred_element_type=jnp.float32)


@jax.jit
def _tc_epilogue(sC, kC, sD, kD, sM, kM, w1, w2, w3):
    R = 640
    grid = (NPAD // R,)
    s_spec = pl.BlockSpec((NC, R, D), lambda i: (0, i, 0))
    k_spec = pl.BlockSpec((R, 1), lambda i: (i, 0))
    w_spec = pl.BlockSpec((D, D), lambda i: (0, 0))
    w2_spec = pl.BlockSpec((D, 1), lambda i: (0, 0))
    return pl.pallas_call(
        _tc_body,
        grid=grid,
        in_specs=[s_spec, k_spec, s_spec, k_spec, s_spec, k_spec,
                  w_spec, w2_spec, w_spec],
        out_specs=pl.BlockSpec((R, D), lambda i: (i, 0)),
        out_shape=jax.ShapeDtypeStruct((NPAD, D), jnp.float32),
    )(sC, kC, sD, kD, sM, kM, w1, w2, w3)


def kernel(E_C, E_D, E_M, dst_C, dst_D, dst_M, W1, W2, W3):
    sC, hC, sD, hD, sM, hM = _sc_segment_sums(
        E_C, dst_C, E_D, dst_D, E_M, dst_M)
    kC = hC.sum(axis=0).reshape(NPAD, 1)
    kD = hD.sum(axis=0).reshape(NPAD, 1)
    kM = hM.sum(axis=0).reshape(NPAD, 1)
    return _tc_epilogue(sC, kC, sD, kD, sM, kM, W1, W2, W3)[:N_NODES]


# speedup vs baseline: 6.5178x; 1.3608x over previous
import functools

import jax
import jax.numpy as jnp
from jax import lax
from jax.experimental import pallas as pl
from jax.experimental.pallas import tpu as pltpu
from jax.experimental.pallas import tpu_sc as plsc

N_NODES = 10000
N_EDGES = 320000
D = 128

NC = 2
NS = 16
NW = NC * NS
GROUP = 80
NG = N_EDGES // GROUP
TRIPS = NG // NW
PAIRS = (TRIPS - 1) // 2
NPAD = 10240
NPT = NPAD // NS


def _sc_body(eC, dC, eD, dD, eM, dM,
             sC, hC, sD, hD, sM, hM,
             acc, eb0, eb1, ib0, ib1, hist,
             semE0, semE1, semI0, semI1):
    c = lax.axis_index("c")
    s = lax.axis_index("s")
    wid = s * NC + c
    row0 = s * NPT

    for (E, Dst, S_out, H_out) in ((eC, dC, sC, hC),
                                   (eD, dD, sD, hD),
                                   (eM, dM, sM, hM)):
        def _zero_row(i, carry):
            for j in range(D // 16):
                eb0[i, pl.ds(j * 16, 16)] = jnp.zeros((16,), jnp.float32)
            return carry
        lax.fori_loop(0, GROUP, _zero_row, 0)
        for z in range(NPT // GROUP):
            pltpu.sync_copy(eb0, acc.at[pl.ds(row0 + z * GROUP, GROUP)])

        def _zero_hist(i, carry):
            hist[pl.ds(i * 16, 16)] = jnp.zeros((16,), jnp.float32)
            return carry
        lax.fori_loop(0, NPAD // 16, _zero_hist, 0)
        plsc.subcore_barrier()

        def _start(t, ebuf, ibuf, semE, semI):
            g = wid + NW * t
            pltpu.async_copy(E.at[pl.ds(g * GROUP, GROUP)], ebuf, semE)
            pltpu.async_copy(Dst.at[pl.ds(g * GROUP, GROUP)], ibuf, semI)

        def _wait(t, ebuf, ibuf, semE, semI):
            g = wid + NW * t
            pltpu.make_async_copy(E.at[pl.ds(g * GROUP, GROUP)], ebuf,
                                  semE).wait()
            pltpu.make_async_copy(Dst.at[pl.ds(g * GROUP, GROUP)], ibuf,
                                  semI).wait()

        def _consume(ebuf, ibuf):
            pltpu.sync_copy(ebuf, acc.at[ibuf], add=True)
            for j in range(GROUP // 16):
                idx = ibuf[pl.ds(j * 16, 16)]
                plsc.addupdate_scatter(
                    hist, [idx], jnp.ones((16,), jnp.float32))

        _start(0, eb0, ib0, semE0, semI0)

        def _pair(p, carry):
            t0 = 2 * p
            _wait(t0, eb0, ib0, semE0, semI0)
            _start(t0 + 1, eb1, ib1, semE1, semI1)
            _consume(eb0, ib0)
            _wait(t0 + 1, eb1, ib1, semE1, semI1)
            _start(t0 + 2, eb0, ib0, semE0, semI0)
            _consume(eb1, ib1)
            return carry
        lax.fori_loop(0, PAIRS, _pair, 0)
        _wait(TRIPS - 1, eb0, ib0, semE0, semI0)
        _consume(eb0, ib0)
        plsc.subcore_barrier()

        nz = NPT // GROUP
        for z in range(nz):
            r0 = row0 + z * GROUP
            ebuf, semE = (eb0, semE0) if z % 2 == 0 else (eb1, semE1)
            if z >= 2:
                rp = row0 + (z - 2) * GROUP
                pltpu.make_async_copy(
                    ebuf, S_out.at[c, pl.ds(rp, GROUP)], semE).wait()
            pltpu.sync_copy(acc.at[pl.ds(r0, GROUP)], ebuf)
            pltpu.async_copy(ebuf, S_out.at[c, pl.ds(r0, GROUP)], semE)
        for z in (nz - 2, nz - 1):
            r0 = row0 + z * GROUP
            ebuf, semE = (eb0, semE0) if z % 2 == 0 else (eb1, semE1)
            pltpu.make_async_copy(
                ebuf, S_out.at[c, pl.ds(r0, GROUP)], semE).wait()
        pltpu.sync_copy(hist, H_out.at[wid])


@jax.jit
def _sc_segment_sums(eC, dC, eD, dD, eM, dM):
    out_type = [
        jax.ShapeDtypeStruct((NC, NPAD, D), jnp.float32),
        jax.ShapeDtypeStruct((NW, NPAD), jnp.float32),
    ] * 3
    scratch = [
        pltpu.VMEM_SHARED((NPAD, D), jnp.float32),
        pltpu.VMEM((GROUP, D), jnp.float32),
        pltpu.VMEM((GROUP, D), jnp.float32),
        pltpu.VMEM((GROUP,), jnp.int32),
        pltpu.VMEM((GROUP,), jnp.int32),
        pltpu.VMEM((NPAD,), jnp.float32),
        pltpu.SemaphoreType.DMA,
        pltpu.SemaphoreType.DMA,
        pltpu.SemaphoreType.DMA,
        pltpu.SemaphoreType.DMA,
    ]
    mesh = plsc.VectorSubcoreMesh(core_axis_name="c", subcore_axis_name="s")
    f = pl.kernel(_sc_body, out_type=out_type, mesh=mesh,
                  scratch_types=scratch,
                  compiler_params=pltpu.CompilerParams(
                      needs_layout_passes=False))
    return f(eC, dC, eD, dD, eM, dM)


def _tc_body(sC, kC, sD, kD, sM, kM, w1, w2, w3, out):
    w1v = w1[...]
    w2v = w2[...]
    w3v = w3[...]

    def seg_mean(sref, kref):
        ssum = sref[0] + sref[1]
        return ssum / jnp.maximum(kref[...], 1.0)

    yC = seg_mean(sC, kC)
    yD = seg_mean(sD, kD)
    yM = seg_mean(sM, kM)

    def att(y):
        return jnp.dot(jnp.tanh(jnp.dot(y, w1v, preferred_element_type=jnp.float32)),
                       w2v, preferred_element_type=jnp.float32)

    eC = att(yC)
    eD = att(yD)
    eM = att(yM)
    m = jnp.maximum(jnp.maximum(eC, eD), eM)
    xC = jnp.exp(eC - m)
    xD = jnp.exp(eD - m)
    xM = jnp.exp(eM - m)
    den = xC + xD + xM
    oy = (xC * yC + xD * yD + xM * yM) / den
    out[...] = jnp.dot(oy, w3v, preferred_element_type=jnp.float32)


@jax.jit
def _tc_epilogue(sC, kC, sD, kD, sM, kM, w1, w2, w3):
    R = 640
    grid = (NPAD // R,)
    s_spec = pl.BlockSpec((NC, R, D), lambda i: (0, i, 0))
    k_spec = pl.BlockSpec((R, 1), lambda i: (i, 0))
    w_spec = pl.BlockSpec((D, D), lambda i: (0, 0))
    w2_spec = pl.BlockSpec((D, 1), lambda i: (0, 0))
    return pl.pallas_call(
        _tc_body,
        grid=grid,
        in_specs=[s_spec, k_spec, s_spec, k_spec, s_spec, k_spec,
                  w_spec, w2_spec, w_spec],
        out_specs=pl.BlockSpec((R, D), lambda i: (i, 0)),
        out_shape=jax.ShapeDtypeStruct((NPAD, D), jnp.float32),
    )(sC, kC, sD, kD, sM, kM, w1, w2, w3)


def kernel(E_C, E_D, E_M, dst_C, dst_D, dst_M, W1, W2, W3):
    sC, hC, sD, hD, sM, hM = _sc_segment_sums(
        E_C, dst_C, E_D, dst_D, E_M, dst_M)
    kC = hC.sum(axis=0).reshape(NPAD, 1)
    kD = hD.sum(axis=0).reshape(NPAD, 1)
    kM = hM.sum(axis=0).reshape(NPAD, 1)
    return _tc_epilogue(sC, kC, sD, kD, sM, kM, W1, W2, W3)[:N_NODES]


# speedup vs baseline: 9.4739x; 1.4535x over previous
import functools

import jax
import jax.numpy as jnp
from jax import lax
from jax.experimental import pallas as pl
from jax.experimental.pallas import tpu as pltpu
from jax.experimental.pallas import tpu_sc as plsc

N_NODES = 10000
N_EDGES = 320000
D = 128

NC = 2
NS = 16
NW = NC * NS
GROUP = 80
NG = N_EDGES // GROUP
TRIPS = NG // NW
PAIRS = (TRIPS - 1) // 2
NPAD = 10240
NPT = NPAD // NS


def _sc_body(eC, dC, eD, dD, eM, dM,
             sC, hC, sD, hD, sM, hM,
             acc, eb0, eb1, eb2, ib0, ib1, ib2, hist,
             semE0, semE1, semE2, semI0, semI1, semI2,
             semS0, semS1, semS2):
    c = lax.axis_index("c")
    s = lax.axis_index("s")
    wid = s * NC + c
    row0 = s * NPT

    for (E, Dst, S_out, H_out) in ((eC, dC, sC, hC),
                                   (eD, dD, sD, hD),
                                   (eM, dM, sM, hM)):
        def _zero_row(i, carry):
            for j in range(D // 16):
                eb0[i, pl.ds(j * 16, 16)] = jnp.zeros((16,), jnp.float32)
            return carry
        lax.fori_loop(0, GROUP, _zero_row, 0)
        for z in range(NPT // GROUP):
            pltpu.sync_copy(eb0, acc.at[pl.ds(row0 + z * GROUP, GROUP)])

        def _zero_hist(i, carry):
            hist[pl.ds(i * 16, 16)] = jnp.zeros((16,), jnp.float32)
            return carry
        lax.fori_loop(0, NPAD // 16, _zero_hist, 0)
        plsc.subcore_barrier()

        slots = ((eb0, ib0, semE0, semI0, semS0),
                 (eb1, ib1, semE1, semI1, semS1),
                 (eb2, ib2, semE2, semI2, semS2))

        def _g_start(t, sl):
            g = wid + NW * t
            pltpu.async_copy(E.at[pl.ds(g * GROUP, GROUP)], sl[0], sl[2])
            pltpu.async_copy(Dst.at[pl.ds(g * GROUP, GROUP)], sl[1], sl[3])

        def _g_wait(t, sl):
            g = wid + NW * t
            pltpu.make_async_copy(E.at[pl.ds(g * GROUP, GROUP)], sl[0],
                                  sl[2]).wait()
            pltpu.make_async_copy(Dst.at[pl.ds(g * GROUP, GROUP)], sl[1],
                                  sl[3]).wait()

        def _s_start(sl):
            pltpu.async_copy(sl[0], acc.at[sl[1]], sl[4], add=True)

        def _s_wait(sl):
            pltpu.make_async_copy(sl[0], acc.at[sl[1]], sl[4]).wait()

        def _hist(sl):
            for j in range(GROUP // 16):
                idx = sl[1][pl.ds(j * 16, 16)]
                plsc.addupdate_scatter(
                    hist, [idx], jnp.ones((16,), jnp.float32))

        _g_start(0, slots[0])
        _g_start(1, slots[1])
        _g_wait(0, slots[0])
        _s_start(slots[0])
        _hist(slots[0])
        _g_start(2, slots[2])
        _g_wait(1, slots[1])
        _s_start(slots[1])
        _hist(slots[1])
        _s_wait(slots[0])
        _g_start(3, slots[0])
        _g_wait(2, slots[2])
        _s_start(slots[2])
        _hist(slots[2])
        _s_wait(slots[1])
        _g_start(4, slots[1])

        def _trip(p, carry):
            for k in range(3):
                t = 3 * p + k
                sl = slots[k]
                nxt = slots[(k + 2) % 3]
                _g_wait(t, sl)
                _s_start(sl)
                _hist(sl)
                _s_wait(nxt)
                _g_start(t + 2, nxt)
            return carry
        lax.fori_loop(1, (TRIPS - 2) // 3, _trip, 0)
        t = TRIPS - 2
        sl = slots[t % 3]
        nxt = slots[(t + 2) % 3]
        _g_wait(t, sl)
        _s_start(sl)
        _hist(sl)
        _s_wait(nxt)
        t = TRIPS - 1
        sl = slots[t % 3]
        nxt = slots[(t + 2) % 3]
        _g_wait(t, sl)
        _s_start(sl)
        _hist(sl)
        _s_wait(nxt)
        _s_wait(slots[(TRIPS - 1) % 3])
        plsc.subcore_barrier()

        nz = NPT // GROUP
        for z in range(nz):
            r0 = row0 + z * GROUP
            ebuf, semE = (eb0, semE0) if z % 2 == 0 else (eb1, semE1)
            if z >= 2:
                rp = row0 + (z - 2) * GROUP
                pltpu.make_async_copy(
                    ebuf, S_out.at[c, pl.ds(rp, GROUP)], semE).wait()
            pltpu.sync_copy(acc.at[pl.ds(r0, GROUP)], ebuf)
            pltpu.async_copy(ebuf, S_out.at[c, pl.ds(r0, GROUP)], semE)
        for z in (nz - 2, nz - 1):
            r0 = row0 + z * GROUP
            ebuf, semE = (eb0, semE0) if z % 2 == 0 else (eb1, semE1)
            pltpu.make_async_copy(
                ebuf, S_out.at[c, pl.ds(r0, GROUP)], semE).wait()
        pltpu.sync_copy(hist, H_out.at[wid])


@jax.jit
def _sc_segment_sums(eC, dC, eD, dD, eM, dM):
    out_type = [
        jax.ShapeDtypeStruct((NC, NPAD, D), jnp.float32),
        jax.ShapeDtypeStruct((NW, NPAD), jnp.float32),
    ] * 3
    scratch = [
        pltpu.VMEM_SHARED((NPAD, D), jnp.float32),
        pltpu.VMEM((GROUP, D), jnp.float32),
        pltpu.VMEM((GROUP, D), jnp.float32),
        pltpu.VMEM((GROUP, D), jnp.float32),
        pltpu.VMEM((GROUP,), jnp.int32),
        pltpu.VMEM((GROUP,), jnp.int32),
        pltpu.VMEM((GROUP,), jnp.int32),
        pltpu.VMEM((NPAD,), jnp.float32),
    ] + [pltpu.SemaphoreType.DMA] * 9
    mesh = plsc.VectorSubcoreMesh(core_axis_name="c", subcore_axis_name="s")
    f = pl.kernel(_sc_body, out_type=out_type, mesh=mesh,
                  scratch_types=scratch,
                  compiler_params=pltpu.CompilerParams(
                      needs_layout_passes=False))
    return f(eC, dC, eD, dD, eM, dM)


def _tc_body(sC, kC, sD, kD, sM, kM, w1, w2, w3, out):
    w1v = w1[...]
    w2v = w2[...]
    w3v = w3[...]

    def seg_mean(sref, kref):
        ssum = sref[0] + sref[1]
        return ssum / jnp.maximum(kref[...], 1.0)

    yC = seg_mean(sC, kC)
    yD = seg_mean(sD, kD)
    yM = seg_mean(sM, kM)

    def att(y):
        return jnp.dot(jnp.tanh(jnp.dot(y, w1v, preferred_element_type=jnp.float32)),
                       w2v, preferred_element_type=jnp.float32)

    eC = att(yC)
    eD = att(yD)
    eM = att(yM)
    m = jnp.maximum(jnp.maximum(eC, eD), eM)
    xC = jnp.exp(eC - m)
    xD = jnp.exp(eD - m)
    xM = jnp.exp(eM - m)
    den = xC + xD + xM
    oy = (xC * yC + xD * yD + xM * yM) / den
    out[...] = jnp.dot(oy, w3v, preferred_element_type=jnp.float32)


@jax.jit
def _tc_epilogue(sC, kC, sD, kD, sM, kM, w1, w2, w3):
    R = 640
    grid = (NPAD // R,)
    s_spec = pl.BlockSpec((NC, R, D), lambda i: (0, i, 0))
    k_spec = pl.BlockSpec((R, 1), lambda i: (i, 0))
    w_spec = pl.BlockSpec((D, D), lambda i: (0, 0))
    w2_spec = pl.BlockSpec((D, 1), lambda i: (0, 0))
    return pl.pallas_call(
        _tc_body,
        grid=grid,
        in_specs=[s_spec, k_spec, s_spec, k_spec, s_spec, k_spec,
                  w_spec, w2_spec, w_spec],
        out_specs=pl.BlockSpec((R, D), lambda i: (i, 0)),
        out_shape=jax.ShapeDtypeStruct((NPAD, D), jnp.float32),
    )(sC, kC, sD, kD, sM, kM, w1, w2, w3)


def kernel(E_C, E_D, E_M, dst_C, dst_D, dst_M, W1, W2, W3):
    sC, hC, sD, hD, sM, hM = _sc_segment_sums(
        E_C, dst_C, E_D, dst_D, E_M, dst_M)
    kC = hC.sum(axis=0).reshape(NPAD, 1)
    kD = hD.sum(axis=0).reshape(NPAD, 1)
    kM = hM.sum(axis=0).reshape(NPAD, 1)
    return _tc_epilogue(sC, kC, sD, kD, sM, kM, W1, W2, W3)[:N_NODES]


# speedup vs baseline: 9.7224x; 1.0262x over previous
import functools

import jax
import jax.numpy as jnp
from jax import lax
from jax.experimental import pallas as pl
from jax.experimental.pallas import tpu as pltpu
from jax.experimental.pallas import tpu_sc as plsc

N_NODES = 10000
N_EDGES = 320000
D = 128

NC = 2
NS = 16
NW = NC * NS
GROUP = 80
NG = N_EDGES // GROUP
TRIPS = NG // NW
PAIRS = (TRIPS - 1) // 2
NPAD = 10240
NPT = NPAD // NS


def _sc_body(eC, dC, eD, dD, eM, dM,
             sC, hC, sD, hD, sM, hM,
             acc, eb0, eb1, eb2, ib0, ib1, ib2, hist,
             semE0, semE1, semE2, semI0, semI1, semI2,
             semS0, semS1, semS2):
    c = lax.axis_index("c")
    s = lax.axis_index("s")
    wid = s * NC + c
    row0 = s * NPT

    for (E, Dst, S_out, H_out) in ((eC, dC, sC, hC),
                                   (eD, dD, sD, hD),
                                   (eM, dM, sM, hM)):
        def _zero_row(i, carry):
            for j in range(D // 16):
                eb2[i, pl.ds(j * 16, 16)] = jnp.zeros((16,), jnp.float32)
                hist[i, pl.ds(j * 16, 16)] = jnp.zeros((16,), jnp.float32)
            return carry
        lax.fori_loop(0, GROUP, _zero_row, 0)
        for z in range(NPT // GROUP):
            sem = semS0 if z % 2 == 0 else semS1
            pltpu.async_copy(eb2, acc.at[pl.ds(row0 + z * GROUP, GROUP)],
                             sem)
        for z in range(NPT // GROUP):
            sem = semS0 if z % 2 == 0 else semS1
            pltpu.make_async_copy(
                eb2, acc.at[pl.ds(row0 + z * GROUP, GROUP)], sem).wait()
        plsc.subcore_barrier()

        slots = ((eb0, ib0, semE0, semI0, semS0),
                 (eb1, ib1, semE1, semI1, semS1),
                 (eb2, ib2, semE2, semI2, semS2))

        def _g_start(t, sl):
            g = wid + NW * t
            pltpu.async_copy(E.at[pl.ds(g * GROUP, GROUP)], sl[0], sl[2])
            pltpu.async_copy(Dst.at[pl.ds(g * GROUP, GROUP)], sl[1], sl[3])

        def _g_wait(t, sl):
            g = wid + NW * t
            pltpu.make_async_copy(E.at[pl.ds(g * GROUP, GROUP)], sl[0],
                                  sl[2]).wait()
            pltpu.make_async_copy(Dst.at[pl.ds(g * GROUP, GROUP)], sl[1],
                                  sl[3]).wait()

        def _s_start(sl):
            pltpu.async_copy(sl[0], acc.at[sl[1]], sl[4], add=True)

        def _s_wait(sl):
            pltpu.make_async_copy(sl[0], acc.at[sl[1]], sl[4]).wait()

        def _hist(sl):
            for j in range(GROUP // 16):
                idx = sl[1][pl.ds(j * 16, 16)]
                plsc.addupdate_scatter(
                    hist,
                    [jax.lax.shift_right_logical(idx, 7),
                     jax.lax.bitwise_and(idx, 127)],
                    jnp.ones((16,), jnp.float32))

        _g_start(0, slots[0])
        _g_start(1, slots[1])
        _g_wait(0, slots[0])
        _s_start(slots[0])
        _hist(slots[0])
        _g_start(2, slots[2])
        _g_wait(1, slots[1])
        _s_start(slots[1])
        _hist(slots[1])
        _s_wait(slots[0])
        _g_start(3, slots[0])
        _g_wait(2, slots[2])
        _s_start(slots[2])
        _hist(slots[2])
        _s_wait(slots[1])
        _g_start(4, slots[1])

        def _trip(p, carry):
            for k in range(3):
                t = 3 * p + k
                sl = slots[k]
                nxt = slots[(k + 2) % 3]
                _g_wait(t, sl)
                _s_start(sl)
                _hist(sl)
                _s_wait(nxt)
                _g_start(t + 2, nxt)
            return carry
        lax.fori_loop(1, (TRIPS - 2) // 3, _trip, 0)
        t = TRIPS - 2
        sl = slots[t % 3]
        nxt = slots[(t + 2) % 3]
        _g_wait(t, sl)
        _s_start(sl)
        _hist(sl)
        _s_wait(nxt)
        t = TRIPS - 1
        sl = slots[t % 3]
        nxt = slots[(t + 2) % 3]
        _g_wait(t, sl)
        _s_start(sl)
        _hist(sl)
        _s_wait(nxt)
        _s_wait(slots[(TRIPS - 1) % 3])
        plsc.subcore_barrier()

        nz = NPT // GROUP
        for z in range(nz):
            r0 = row0 + z * GROUP
            ebuf, semE = (eb0, semE0) if z % 2 == 0 else (eb1, semE1)
            if z >= 2:
                rp = row0 + (z - 2) * GROUP
                pltpu.make_async_copy(
                    ebuf, S_out.at[c, pl.ds(rp, GROUP)], semE).wait()
            pltpu.sync_copy(acc.at[pl.ds(r0, GROUP)], ebuf)
            pltpu.async_copy(ebuf, S_out.at[c, pl.ds(r0, GROUP)], semE)
        for z in (nz - 2, nz - 1):
            r0 = row0 + z * GROUP
            ebuf, semE = (eb0, semE0) if z % 2 == 0 else (eb1, semE1)
            pltpu.make_async_copy(
                ebuf, S_out.at[c, pl.ds(r0, GROUP)], semE).wait()
        pltpu.sync_copy(hist, H_out.at[wid])


@jax.jit
def _sc_segment_sums(eC, dC, eD, dD, eM, dM):
    out_type = [
        jax.ShapeDtypeStruct((NC, NPAD, D), jnp.float32),
        jax.ShapeDtypeStruct((NW, GROUP, D), jnp.float32),
    ] * 3
    scratch = [
        pltpu.VMEM_SHARED((NPAD, D), jnp.float32),
        pltpu.VMEM((GROUP, D), jnp.float32),
        pltpu.VMEM((GROUP, D), jnp.float32),
        pltpu.VMEM((GROUP, D), jnp.float32),
        pltpu.VMEM((GROUP,), jnp.int32),
        pltpu.VMEM((GROUP,), jnp.int32),
        pltpu.VMEM((GROUP,), jnp.int32),
        pltpu.VMEM((GROUP, D), jnp.float32),
    ] + [pltpu.SemaphoreType.DMA] * 9
    mesh = plsc.VectorSubcoreMesh(core_axis_name="c", subcore_axis_name="s")
    f = pl.kernel(_sc_body, out_type=out_type, mesh=mesh,
                  scratch_types=scratch,
                  compiler_params=pltpu.CompilerParams(
                      needs_layout_passes=False))
    return f(eC, dC, eD, dD, eM, dM)


def _tc_body(sC, kC, sD, kD, sM, kM, w1, w2, w3, out):
    w1v = w1[...]
    w2v = w2[...]
    w3v = w3[...]

    def seg_mean(sref, kref):
        ssum = sref[0] + sref[1]
        return ssum / jnp.maximum(kref[...], 1.0)

    yC = seg_mean(sC, kC)
    yD = seg_mean(sD, kD)
    yM = seg_mean(sM, kM)

    def att(y):
        return jnp.dot(jnp.tanh(jnp.dot(y, w1v, preferred_element_type=jnp.float32)),
                       w2v, preferred_element_type=jnp.float32)

    eC = att(yC)
    eD = att(yD)
    eM = att(yM)
    m = jnp.maximum(jnp.maximum(eC, eD), eM)
    xC = jnp.exp(eC - m)
    xD = jnp.exp(eD - m)
    xM = jnp.exp(eM - m)
    den = xC + xD + xM
    oy = (xC * yC + xD * yD + xM * yM) / den
    out[...] = jnp.dot(oy, w3v, preferred_element_type=jnp.float32)


@jax.jit
def _tc_epilogue(sC, kC, sD, kD, sM, kM, w1, w2, w3):
    R = 640
    grid = (NPAD // R,)
    s_spec = pl.BlockSpec((NC, R, D), lambda i: (0, i, 0))
    k_spec = pl.BlockSpec((R, 1), lambda i: (i, 0))
    w_spec = pl.BlockSpec((D, D), lambda i: (0, 0))
    w2_spec = pl.BlockSpec((D, 1), lambda i: (0, 0))
    return pl.pallas_call(
        _tc_body,
        grid=grid,
        in_specs=[s_spec, k_spec, s_spec, k_spec, s_spec, k_spec,
                  w_spec, w2_spec, w_spec],
        out_specs=pl.BlockSpec((R, D), lambda i: (i, 0)),
        out_shape=jax.ShapeDtypeStruct((NPAD, D), jnp.float32),
    )(sC, kC, sD, kD, sM, kM, w1, w2, w3)


def kernel(E_C, E_D, E_M, dst_C, dst_D, dst_M, W1, W2, W3):
    sC, hC, sD, hD, sM, hM = _sc_segment_sums(
        E_C, dst_C, E_D, dst_D, E_M, dst_M)
    kC = hC.sum(axis=0).reshape(NPAD, 1)
    kD = hD.sum(axis=0).reshape(NPAD, 1)
    kM = hM.sum(axis=0).reshape(NPAD, 1)
    return _tc_epilogue(sC, kC, sD, kD, sM, kM, W1, W2, W3)[:N_NODES]
